# baseline (speedup 1.0000x reference)
import functools

import jax
import jax.numpy as jnp
from jax import lax
from jax.experimental import pallas as pl
from jax.experimental.pallas import tpu as pltpu
from jax.experimental.pallas import tpu_sc as plsc

VOCAB = 50000
NQ = 4
NB = 128
LANES = 16
COLV = NB // LANES
NSH = 8
BIG = 2**30

VT = 22528
VCT = 2048
NJT = VT // VCT

SCV = VOCAB - VT
SHARD = 3456
CH = 448
NCHK = -(-SHARD // CH)


def _sc_body(x_hbm, iout_hbm, vout_hbm, bufa, bufb, stage_m, stage_i,
             merge_m, merge_i, shm, shi, sema, semb):
    c = lax.axis_index("c")
    s = lax.axis_index("s")
    q = c * 2 + s // NSH
    w = s % NSH
    v0 = VT + w * SHARD
    limit = jnp.clip(VOCAB - v0, 0, SHARD)
    bufs = (bufa, bufb)
    sems = (sema, semb)

    def chunk_copy(k):
        return pltpu.async_copy(
            x_hbm.at[q, pl.ds(v0 + k * CH, CH), :], bufs[k % 2], sems[k % 2])

    neg = jnp.full((LANES,), -jnp.inf, jnp.float32)
    zero = jnp.zeros((LANES,), jnp.int32)
    ms = [neg] * COLV
    mis = [zero] * COLV

    pending = [chunk_copy(0), chunk_copy(1)]
    for k in range(NCHK):
        pending[k % 2].wait()
        buf = bufs[k % 2]
        base = v0 + k * CH

        def row_body(vr, carry):
            ms, mis = list(carry[0]), list(carry[1])
            vsp = jnp.full((LANES,), base + vr, jnp.int32)
            for u in range(COLV):
                v = buf[vr, pl.ds(u * LANES, LANES)]
                upd = v > ms[u]
                ms[u] = jnp.where(upd, v, ms[u])
                mis[u] = jnp.where(upd, vsp, mis[u])
            return tuple(ms), tuple(mis)

        nrows = jnp.clip(limit - k * CH, 0, CH)
        ms, mis = lax.fori_loop(0, nrows, row_body, (tuple(ms), tuple(mis)))
        ms, mis = list(ms), list(mis)
        if k + 2 < NCHK:
            pending[k % 2] = chunk_copy(k + 2)

    for u in range(COLV):
        stage_m[pl.ds(u * LANES, LANES)] = ms[u]
        stage_i[pl.ds(u * LANES, LANES)] = mis[u]
    pltpu.sync_copy(stage_m, shm.at[s])
    pltpu.sync_copy(stage_i, shi.at[s])
    plsc.subcore_barrier()

    @pl.when(w == 0)
    def _():
        msf = list(ms)
        misf = list(mis)
        for sh in range(1, NSH):
            pltpu.sync_copy(shm.at[s + sh], merge_m)
            pltpu.sync_copy(shi.at[s + sh], merge_i)
            for u in range(COLV):
                mv = merge_m[pl.ds(u * LANES, LANES)]
                iv = merge_i[pl.ds(u * LANES, LANES)]
                take = mv > msf[u]
                msf[u] = jnp.where(take, mv, msf[u])
                misf[u] = jnp.where(take, iv, misf[u])
        for u in range(COLV):
            stage_i[pl.ds(u * LANES, LANES)] = misf[u]
            stage_m[pl.ds(u * LANES, LANES)] = msf[u]
        pltpu.sync_copy(stage_i, iout_hbm.at[pl.ds(q * NB, NB)])
        pltpu.sync_copy(stage_m, vout_hbm.at[pl.ds(q * NB, NB)])


def _tc_body(x_ref, vout_ref, iout_ref):
    j = pl.program_id(1)
    v = x_ref[0]
    cm = jnp.max(v, axis=0)
    ci = jnp.argmax(v, axis=0).astype(jnp.int32) + j * VCT

    @pl.when(j == 0)
    def _():
        vout_ref[0, 0] = cm
        iout_ref[0, 0] = ci

    @pl.when(j > 0)
    def _():
        rm = vout_ref[0, 0]
        upd = cm > rm
        vout_ref[0, 0] = jnp.where(upd, cm, rm)
        iout_ref[0, 0] = jnp.where(upd, ci, iout_ref[0, 0])


def kernel(logits):
    xt = jnp.transpose(logits, (1, 2, 0))
    mesh = plsc.VectorSubcoreMesh(core_axis_name="c", subcore_axis_name="s")
    sc_i, sc_v = pl.kernel(
        _sc_body,
        out_type=(jax.ShapeDtypeStruct((NQ * NB,), jnp.int32),
                  jax.ShapeDtypeStruct((NQ * NB,), jnp.float32)),
        mesh=mesh,
        compiler_params=pltpu.CompilerParams(needs_layout_passes=False),
        scratch_types=[
            pltpu.VMEM((CH, NB), jnp.float32),
            pltpu.VMEM((CH, NB), jnp.float32),
            pltpu.VMEM((NB,), jnp.float32),
            pltpu.VMEM((NB,), jnp.int32),
            pltpu.VMEM((NB,), jnp.float32),
            pltpu.VMEM((NB,), jnp.int32),
            pltpu.VMEM_SHARED((16, NB), jnp.float32),
            pltpu.VMEM_SHARED((16, NB), jnp.int32),
            pltpu.SemaphoreType.DMA,
            pltpu.SemaphoreType.DMA,
        ],
    )(xt)

    tc_v, tc_i = pl.pallas_call(
        _tc_body,
        grid=(NQ, NJT),
        in_specs=[pl.BlockSpec((1, VCT, NB), lambda q, j: (q, j, 0))],
        out_specs=[pl.BlockSpec((1, 1, NB), lambda q, j: (q, 0, 0)),
                   pl.BlockSpec((1, 1, NB), lambda q, j: (q, 0, 0))],
        out_shape=[jax.ShapeDtypeStruct((NQ, 1, NB), jnp.float32),
                   jax.ShapeDtypeStruct((NQ, 1, NB), jnp.int32)],
        compiler_params=pltpu.CompilerParams(
            dimension_semantics=("parallel", "arbitrary")),
    )(xt)

    tc_v = tc_v.reshape(NQ, NB)
    tc_i = tc_i.reshape(NQ, NB)
    sc_i = sc_i.reshape(NQ, NB)
    sc_v = sc_v.reshape(NQ, NB)
    idx = jnp.where(tc_v >= sc_v, tc_i, sc_i)
    return idx.T


# speedup vs baseline: 1.0474x; 1.0474x over previous
import functools

import jax
import jax.numpy as jnp
from jax import lax
from jax.experimental import pallas as pl
from jax.experimental.pallas import tpu as pltpu
from jax.experimental.pallas import tpu_sc as plsc

VOCAB = 50000
NQ = 4
NB = 128
LANES = 16
COLV = NB // LANES
NSH = 8
BIG = 2**30

VT = 24576
VCT = 4096
NJT = VT // VCT

SCV = VOCAB - VT
SHARD = 3200
CH = 448
NCHK = -(-SHARD // CH)


def _sc_body(x_hbm, iout_hbm, vout_hbm, bufa, bufb, stage_m, stage_i,
             merge_m, merge_i, shm, shi, sema, semb):
    c = lax.axis_index("c")
    s = lax.axis_index("s")
    q = c * 2 + s // NSH
    w = s % NSH
    v0 = VT + w * SHARD
    limit = jnp.clip(VOCAB - v0, 0, SHARD)
    bufs = (bufa, bufb)
    sems = (sema, semb)

    def chunk_copy(k):
        return pltpu.async_copy(
            x_hbm.at[q, pl.ds(v0 + k * CH, CH), :], bufs[k % 2], sems[k % 2])

    neg = jnp.full((LANES,), -jnp.inf, jnp.float32)
    zero = jnp.zeros((LANES,), jnp.int32)
    ms = [neg] * COLV
    mis = [zero] * COLV

    pending = [chunk_copy(0), chunk_copy(1)]
    for k in range(NCHK):
        pending[k % 2].wait()
        buf = bufs[k % 2]
        base = v0 + k * CH

        def row_body(vr, carry):
            ms, mis = list(carry[0]), list(carry[1])
            vsp = jnp.full((LANES,), base + vr, jnp.int32)
            for u in range(COLV):
                v = buf[vr, pl.ds(u * LANES, LANES)]
                upd = v > ms[u]
                ms[u] = jnp.where(upd, v, ms[u])
                mis[u] = jnp.where(upd, vsp, mis[u])
            return tuple(ms), tuple(mis)

        nrows = jnp.clip(limit - k * CH, 0, CH)
        ms, mis = lax.fori_loop(0, nrows, row_body, (tuple(ms), tuple(mis)))
        ms, mis = list(ms), list(mis)
        if k + 2 < NCHK:
            pending[k % 2] = chunk_copy(k + 2)

    for u in range(COLV):
        stage_m[pl.ds(u * LANES, LANES)] = ms[u]
        stage_i[pl.ds(u * LANES, LANES)] = mis[u]
    pltpu.sync_copy(stage_m, shm.at[s])
    pltpu.sync_copy(stage_i, shi.at[s])
    plsc.subcore_barrier()

    @pl.when(w == 0)
    def _():
        msf = list(ms)
        misf = list(mis)
        for sh in range(1, NSH):
            pltpu.sync_copy(shm.at[s + sh], merge_m)
            pltpu.sync_copy(shi.at[s + sh], merge_i)
            for u in range(COLV):
                mv = merge_m[pl.ds(u * LANES, LANES)]
                iv = merge_i[pl.ds(u * LANES, LANES)]
                take = mv > msf[u]
                msf[u] = jnp.where(take, mv, msf[u])
                misf[u] = jnp.where(take, iv, misf[u])
        for u in range(COLV):
            stage_i[pl.ds(u * LANES, LANES)] = misf[u]
            stage_m[pl.ds(u * LANES, LANES)] = msf[u]
        pltpu.sync_copy(stage_i, iout_hbm.at[pl.ds(q * NB, NB)])
        pltpu.sync_copy(stage_m, vout_hbm.at[pl.ds(q * NB, NB)])


def _tc_body(x_ref, vout_ref, iout_ref):
    j = pl.program_id(1)
    v = x_ref[0]
    cm = jnp.max(v, axis=0)
    ci = jnp.argmax(v, axis=0).astype(jnp.int32) + j * VCT

    @pl.when(j == 0)
    def _():
        vout_ref[0, 0] = cm
        iout_ref[0, 0] = ci

    @pl.when(j > 0)
    def _():
        rm = vout_ref[0, 0]
        upd = cm > rm
        vout_ref[0, 0] = jnp.where(upd, cm, rm)
        iout_ref[0, 0] = jnp.where(upd, ci, iout_ref[0, 0])


def kernel(logits):
    xt = jnp.transpose(logits, (1, 2, 0))
    mesh = plsc.VectorSubcoreMesh(core_axis_name="c", subcore_axis_name="s")
    sc_i, sc_v = pl.kernel(
        _sc_body,
        out_type=(jax.ShapeDtypeStruct((NQ * NB,), jnp.int32),
                  jax.ShapeDtypeStruct((NQ * NB,), jnp.float32)),
        mesh=mesh,
        compiler_params=pltpu.CompilerParams(needs_layout_passes=False),
        scratch_types=[
            pltpu.VMEM((CH, NB), jnp.float32),
            pltpu.VMEM((CH, NB), jnp.float32),
            pltpu.VMEM((NB,), jnp.float32),
            pltpu.VMEM((NB,), jnp.int32),
            pltpu.VMEM((NB,), jnp.float32),
            pltpu.VMEM((NB,), jnp.int32),
            pltpu.VMEM_SHARED((16, NB), jnp.float32),
            pltpu.VMEM_SHARED((16, NB), jnp.int32),
            pltpu.SemaphoreType.DMA,
            pltpu.SemaphoreType.DMA,
        ],
    )(xt)

    tc_v, tc_i = pl.pallas_call(
        _tc_body,
        grid=(NQ, NJT),
        in_specs=[pl.BlockSpec((1, VCT, NB), lambda q, j: (q, j, 0))],
        out_specs=[pl.BlockSpec((1, 1, NB), lambda q, j: (q, 0, 0)),
                   pl.BlockSpec((1, 1, NB), lambda q, j: (q, 0, 0))],
        out_shape=[jax.ShapeDtypeStruct((NQ, 1, NB), jnp.float32),
                   jax.ShapeDtypeStruct((NQ, 1, NB), jnp.int32)],
        compiler_params=pltpu.CompilerParams(
            dimension_semantics=("parallel", "arbitrary")),
    )(xt)

    tc_v = tc_v.reshape(NQ, NB)
    tc_i = tc_i.reshape(NQ, NB)
    sc_i = sc_i.reshape(NQ, NB)
    sc_v = sc_v.reshape(NQ, NB)
    idx = jnp.where(tc_v >= sc_v, tc_i, sc_i)
    return idx.T
